# baseline (speedup 1.0000x reference)
import functools

import jax
import jax.numpy as jnp
from jax import lax
from jax.experimental import pallas as pl
from jax.experimental.pallas import tpu as pltpu
from jax.experimental.pallas import tpu_sc as plsc

N = 10000
E = 160000
G = 64
D = 300
DP = 512
DH = 128
DU = 304
NPAD = 10240
EBR = 2000
NBR = 400
EPS = 1e-5


def _softplus(x):
    return jnp.log1p(jnp.minimum(jnp.exp(x), jnp.float32(1e38)))


def _norm_act(h, stats, gb, nrows):
    s = stats[0, :]
    ss = stats[1, :]
    mean = s / nrows
    var = ss / nrows - mean * mean
    a = lax.rsqrt(var + EPS) * gb[0, :]
    b = gb[1, :] - mean * a
    return _softplus(h * a[None, :] + b[None, :])


def _stats_update(stats_ref, hb, step):
    @pl.when(step == 0)
    def _():
        stats_ref[...] = jnp.zeros_like(stats_ref)

    ones8 = jnp.ones((8, hb.shape[0]), jnp.bfloat16)
    ps = jnp.dot(ones8, hb, preferred_element_type=jnp.float32)
    pss = jnp.dot(ones8, hb * hb, preferred_element_type=jnp.float32)
    pad = jnp.zeros((6, hb.shape[1]), jnp.float32)
    stats_ref[...] += jnp.concatenate([ps[0:1, :], pss[0:1, :], pad], axis=0)


def _rows(br):
    return pl.BlockSpec((br, None), lambda i: (i, 0))


def _full(shape):
    return pl.BlockSpec(shape, lambda i: tuple(0 for _ in shape))


def _acc_spec(shape):
    return pl.BlockSpec(shape, lambda i: tuple(0 for _ in shape))


def _rows_spec(br, ncols):
    return pl.BlockSpec((br, ncols), lambda i: (i, 0))


_GC = 128
_GW = 256


def _sc_gather(table, idx1, idx2):
    nw = 32
    bpw = E // nw
    nfull = bpw // _GC
    npair = nfull // 2
    tail = bpw - nfull * _GC
    mesh = plsc.VectorSubcoreMesh(core_axis_name="c", subcore_axis_name="s")

    @functools.partial(
        pl.kernel,
        mesh=mesh,
        out_type=[
            jax.ShapeDtypeStruct((E, _GW), jnp.int32),
            jax.ShapeDtypeStruct((E, _GW), jnp.int32),
        ],
        scratch_types=[
            pltpu.VMEM((bpw,), jnp.int32),
            pltpu.VMEM((_GC, _GW), jnp.int32),
            pltpu.VMEM((_GC, _GW), jnp.int32),
            pltpu.SemaphoreType.DMA,
            pltpu.SemaphoreType.DMA,
            pltpu.SemaphoreType.DMA,
            pltpu.SemaphoreType.DMA,
        ],
    )
    def k(tab_hbm, i1_hbm, i2_hbm, o1_hbm, o2_hbm,
          idx_v, b0, b1, sg0, sg1, sw0, sw1):
        wid = lax.axis_index("s") * 2 + lax.axis_index("c")
        base = wid * bpw

        def one_table(i_hbm, o_hbm):
            pltpu.sync_copy(i_hbm.at[pl.ds(base, bpw)], idx_v)

            def pair(jj, _):
                j0 = jj * 2
                g0 = pltpu.async_copy(
                    tab_hbm.at[idx_v.at[pl.ds(j0 * _GC, _GC)]], b0, sg0)
                g1 = pltpu.async_copy(
                    tab_hbm.at[idx_v.at[pl.ds((j0 + 1) * _GC, _GC)]], b1, sg1)
                g0.wait()
                w0 = pltpu.async_copy(
                    b0, o_hbm.at[pl.ds(base + j0 * _GC, _GC)], sw0)
                g1.wait()
                w1 = pltpu.async_copy(
                    b1, o_hbm.at[pl.ds(base + (j0 + 1) * _GC, _GC)], sw1)
                w0.wait()
                w1.wait()
                return 0

            lax.fori_loop(0, npair, pair, 0)
            j = nfull - 1
            pltpu.async_copy(
                tab_hbm.at[idx_v.at[pl.ds(j * _GC, _GC)]], b0, sg0).wait()
            w0 = pltpu.async_copy(
                b0, o_hbm.at[pl.ds(base + j * _GC, _GC)], sw0)
            pltpu.async_copy(
                tab_hbm.at[idx_v.at[pl.ds(nfull * _GC, tail)]],
                b1.at[pl.ds(0, tail)], sg1).wait()
            pltpu.sync_copy(b1.at[pl.ds(0, tail)],
                            o_hbm.at[pl.ds(base + nfull * _GC, tail)])
            w0.wait()

        one_table(i1_hbm, o1_hbm)
        one_table(i2_hbm, o2_hbm)

    return k(table, idx1, idx2)


_SCC = 40


def _sc_scatter(et0, et1, et2, idx_p1, idx_p2, zeros_init):
    nt = 16
    epw = E // nt
    nch1 = epw // _SCC
    npr1 = nch1 // 2
    e2w = E // 32
    nch2 = e2w // _SCC
    npr2 = nch2 // 2
    npt = NPAD // nt
    mesh = plsc.VectorSubcoreMesh(core_axis_name="c", subcore_axis_name="s")

    @functools.partial(
        pl.kernel,
        mesh=mesh,
        out_type=[
            jax.ShapeDtypeStruct((NPAD, DH), jnp.float32),
            jax.ShapeDtypeStruct((NPAD, DH), jnp.float32),
            jax.ShapeDtypeStruct((NPAD, DH), jnp.float32),
            jax.ShapeDtypeStruct((NPAD, DH), jnp.float32),
        ],
        scratch_types=[
            pltpu.VMEM((nch1, _SCC), jnp.int32),
            pltpu.VMEM((_SCC, DH), jnp.float32),
            pltpu.VMEM((_SCC, DH), jnp.float32),
            pltpu.VMEM_SHARED((NPAD, DH), jnp.float32),
            pltpu.SemaphoreType.DMA,
            pltpu.SemaphoreType.DMA,
        ],
    )
    def k(e0_hbm, e1_hbm, e2_hbm, ip1_hbm, ip2_hbm, z_hbm,
          o0_hbm, o1_hbm, o2a_hbm, o2b_hbm,
          idx_v, b0, b1, acc, s0, s1):
        cid = lax.axis_index("c")
        sid = lax.axis_index("s")
        r0 = sid * npt
        pltpu.sync_copy(ip1_hbm.at[sid], idx_v)
        pltpu.sync_copy(z_hbm.at[pl.ds(r0, npt)], acc.at[pl.ds(r0, npt)])
        plsc.subcore_barrier()

        def pairs(src_hbm, base, npairs):
            def pair(jj, _):
                j0 = jj * 2
                off0 = base + j0 * _SCC
                l0 = pltpu.async_copy(src_hbm.at[pl.ds(off0, _SCC)], b0, s0)
                l1 = pltpu.async_copy(
                    src_hbm.at[pl.ds(off0 + _SCC, _SCC)], b1, s1)
                l0.wait()
                pltpu.sync_copy(b0, acc.at[idx_v.at[j0]], add=True)
                l1.wait()
                pltpu.sync_copy(b1, acc.at[idx_v.at[j0 + 1]], add=True)
                return 0

            lax.fori_loop(0, npairs, pair, 0)

        @pl.when(cid == 0)
        def _():
            pairs(e0_hbm, sid * epw, npr1)

        @pl.when(cid == 1)
        def _():
            pairs(e1_hbm, sid * epw, npr1)

        plsc.subcore_barrier()

        @pl.when(cid == 0)
        def _():
            pltpu.sync_copy(acc.at[pl.ds(r0, npt)], o0_hbm.at[pl.ds(r0, npt)])

        @pl.when(cid == 1)
        def _():
            pltpu.sync_copy(acc.at[pl.ds(r0, npt)], o1_hbm.at[pl.ds(r0, npt)])

        plsc.subcore_barrier()
        pltpu.sync_copy(ip2_hbm.at[cid * nt + sid], idx_v.at[pl.ds(0, nch2)])
        pltpu.sync_copy(z_hbm.at[pl.ds(r0, npt)], acc.at[pl.ds(r0, npt)])
        plsc.subcore_barrier()

        base2 = cid * (E // 2) + sid * e2w
        pairs(e2_hbm, base2, npr2)
        j = nch2 - 1
        pltpu.sync_copy(e2_hbm.at[pl.ds(base2 + j * _SCC, _SCC)], b0)
        pltpu.sync_copy(b0, acc.at[idx_v.at[j]], add=True)
        plsc.subcore_barrier()

        @pl.when(cid == 0)
        def _():
            pltpu.sync_copy(acc.at[pl.ds(r0, npt)], o2a_hbm.at[pl.ds(r0, npt)])

        @pl.when(cid == 1)
        def _():
            pltpu.sync_copy(acc.at[pl.ds(r0, npt)], o2b_hbm.at[pl.ds(r0, npt)])

    return k(et0, et1, et2, idx_p1, idx_p2, zeros_init)


def _tc_h1(fs, fr, edges, gbond2d, states, wa, wb, wc, wd_uc):
    grid = (E // EBR,)

    def body(fs_ref, fr_ref, e_ref, gb_ref, s_ref, wa_ref, wb_ref, wc_ref,
             wduc_ref, h_ref, st_ref, sn_ref, sb_v):
        i = pl.program_id(0)

        @pl.when(i == 0)
        def _():
            sbsn = jnp.dot(s_ref[...].astype(jnp.bfloat16), wduc_ref[...],
                           preferred_element_type=jnp.float32)
            sb_v[...] = sbsn[:, :600].astype(jnp.bfloat16)
            sn_ref[...] = sbsn[:, 600:].astype(jnp.bfloat16)

        oh = (gb_ref[...] == lax.broadcasted_iota(jnp.int32, (EBR, G), 1))
        h = jnp.dot(e_ref[...].astype(jnp.bfloat16), wc_ref[...],
                    preferred_element_type=jnp.float32)
        h += jnp.dot(oh.astype(jnp.bfloat16), sb_v[...],
                     preferred_element_type=jnp.float32)
        for x_ref, w_ref in ((fs_ref, wa_ref), (fr_ref, wb_ref)):
            x_i = x_ref[...]
            x_lo = lax.bitcast_convert_type(
                jnp.left_shift(x_i, 16), jnp.float32)
            x_hi = lax.bitcast_convert_type(
                jnp.bitwise_and(x_i[:, 0:128], jnp.int32(-65536)), jnp.float32)
            h += jnp.dot(x_lo.astype(jnp.bfloat16), w_ref[pl.ds(0, _GW), :],
                         preferred_element_type=jnp.float32)
            h += jnp.dot(x_hi.astype(jnp.bfloat16), w_ref[pl.ds(_GW, 128), :],
                         preferred_element_type=jnp.float32)
        hb = h.astype(jnp.bfloat16)
        h_ref[...] = hb
        _stats_update(st_ref, hb, i)

    return pl.pallas_call(
        body,
        grid=grid,
        in_specs=[
            _rows_spec(EBR, _GW), _rows_spec(EBR, _GW),
            _rows_spec(EBR, D),
            _rows_spec(EBR, 1),
            _acc_spec((G, D)),
            _acc_spec((DP, 600)), _acc_spec((DP, 600)), _acc_spec((D, 600)),
            _acc_spec((D, 1200)),
        ],
        out_specs=[_rows_spec(EBR, 600), _acc_spec((8, 600)),
                   _acc_spec((G, 600))],
        out_shape=[
            jax.ShapeDtypeStruct((E, 600), jnp.bfloat16),
            jax.ShapeDtypeStruct((8, 600), jnp.float32),
            jax.ShapeDtypeStruct((G, 600), jnp.bfloat16),
        ],
        scratch_shapes=[pltpu.VMEM((G, 600), jnp.bfloat16)],
    )(fs, fr, edges, gbond2d, states, wa, wb, wc, wd_uc)


def _tc_mm(h_prev, stats, gb, w, nrows):
    rtot, kdim = h_prev.shape
    ndim = w.shape[1]
    br = EBR if rtot == E else NBR
    grid = (rtot // br,)

    def body(hp_ref, st_in_ref, gb_ref, w_ref, h_ref, st_ref):
        i = pl.program_id(0)
        x = _norm_act(hp_ref[...].astype(jnp.float32), st_in_ref[...],
                      gb_ref[...], nrows)
        h = jnp.dot(x.astype(jnp.bfloat16), w_ref[...],
                    preferred_element_type=jnp.float32)
        hb = h.astype(jnp.bfloat16)
        h_ref[...] = hb
        _stats_update(st_ref, hb, i)

    return pl.pallas_call(
        body,
        grid=grid,
        in_specs=[
            _rows_spec(br, kdim), _acc_spec((8, kdim)), _acc_spec((8, kdim)),
            _acc_spec((kdim, ndim)),
        ],
        out_specs=[_rows_spec(br, ndim), _acc_spec((8, ndim))],
        out_shape=[
            jax.ShapeDtypeStruct((rtot, ndim), jnp.bfloat16),
            jax.ShapeDtypeStruct((8, ndim), jnp.float32),
        ],
    )(h_prev, stats, gb, w)


def _tc_ekp_h4(h3, stats3, gb3, edges, gbond2d, v0):
    grid = (E // EBR,)

    def body(h3_ref, st3_ref, gb3_ref, e_ref, gbd_ref, v0_ref,
             ekp_ref, h4_ref, st4_ref, ue_ref):
        i = pl.program_id(0)
        ek = e_ref[...] + _norm_act(h3_ref[...].astype(jnp.float32),
                                    st3_ref[...], gb3_ref[...], E)
        ekp_ref[...] = ek
        ekb = ek.astype(jnp.bfloat16)
        h4 = jnp.dot(ekb, v0_ref[...], preferred_element_type=jnp.float32)
        h4b = h4.astype(jnp.bfloat16)
        h4_ref[...] = h4b
        _stats_update(st4_ref, h4b, i)
        oh = (gbd_ref[...] == lax.broadcasted_iota(jnp.int32, (EBR, G), 1))
        ekx = jnp.concatenate(
            [ekb, jnp.ones((EBR, 1), jnp.bfloat16),
             jnp.zeros((EBR, 3), jnp.bfloat16)], axis=1)

        @pl.when(i == 0)
        def _():
            ue_ref[...] = jnp.zeros_like(ue_ref)

        ue_ref[...] += lax.dot_general(
            oh.astype(jnp.bfloat16), ekx, (((0,), (0,)), ((), ())),
            preferred_element_type=jnp.float32)

    return pl.pallas_call(
        body,
        grid=grid,
        in_specs=[
            _rows_spec(EBR, D), _acc_spec((8, D)), _acc_spec((8, D)),
            _rows_spec(EBR, D), _rows_spec(EBR, 1), _acc_spec((D, 600)),
        ],
        out_specs=[_rows_spec(EBR, D), _rows_spec(EBR, 600),
                   _acc_spec((8, 600)), _acc_spec((G, DU))],
        out_shape=[
            jax.ShapeDtypeStruct((E, D), jnp.float32),
            jax.ShapeDtypeStruct((E, 600), jnp.bfloat16),
            jax.ShapeDtypeStruct((8, 600), jnp.float32),
            jax.ShapeDtypeStruct((G, DU), jnp.float32),
        ],
    )(h3, stats3, gb3, edges, gbond2d, v0)


def _tc_et(h5, stats5, gb5):
    grid = (E // EBR,)

    def body(h5_ref, st_ref, gb_ref, s0_ref, s1_ref, s2_ref):
        et = _norm_act(h5_ref[...].astype(jnp.float32), st_ref[...],
                       gb_ref[...], E)
        one = jnp.ones((EBR, 1), jnp.float32)
        zpad = jnp.zeros((EBR, 83), jnp.float32)
        s0_ref[...] = et[:, 0:128]
        s1_ref[...] = et[:, 128:256]
        s2_ref[...] = jnp.concatenate([et[:, 256:300], one, zpad], axis=1)

    return pl.pallas_call(
        body,
        grid=grid,
        in_specs=[_rows_spec(EBR, D), _acc_spec((8, D)), _acc_spec((8, D))],
        out_specs=[_rows_spec(EBR, DH), _rows_spec(EBR, DH),
                   _rows_spec(EBR, DH)],
        out_shape=[
            jax.ShapeDtypeStruct((E, DH), jnp.float32),
            jax.ShapeDtypeStruct((E, DH), jnp.float32),
            jax.ShapeDtypeStruct((E, DH), jnp.float32),
        ],
    )(h5, stats5, gb5)


def _tc_node_all(acc0, acc1, acc2a, acc2b, nodes, gnode2d, ue_acc, states,
                 ua, ub, sn, w2, w3, gb1, gb2, gb3,
                 wu0, wu1, wu2, gu0, gu1, gu2):
    nb = N // NBR
    grid = (4 * nb,)

    def jmap(*phases):
        def f(i):
            p = i // nb
            use = functools.reduce(jnp.logical_or,
                                   [p == q for q in phases])
            return (jnp.where(use, i % nb, 0), 0)
        return f

    def acc_stats(st_v, hb, w):
        ones8 = jnp.ones((8, NBR), jnp.bfloat16)
        ps = jnp.dot(ones8, hb, preferred_element_type=jnp.float32)
        pss = jnp.dot(ones8, hb * hb, preferred_element_type=jnp.float32)
        st_v[...] += jnp.concatenate(
            [ps[0:1, :], pss[0:1, :], jnp.zeros((6, w), jnp.float32)], axis=0)

    def ubn_act(h, gb):
        mean = jnp.mean(h, axis=0)
        var = jnp.mean(h * h, axis=0) - mean * mean
        a = lax.rsqrt(var + EPS) * gb[0:1, :]
        return _softplus(h * a + (gb[1:2, :] - mean[None, :] * a))

    def body(a0, a1, a2a, a2b, n_ref, gn_ref, ue_ref, s_ref,
             ua_ref, ub_ref, sn_ref, w2_ref, w3_ref,
             gb1_ref, gb2_ref, gb3_ref,
             wu0_ref, wu1_ref, wu2_ref, gu0_ref, gu1_ref, gu2_ref,
             v_ref, up_ref,
             hv1_v, hv2_v, st1_v, st2_v, st3_v, uv_v):
        i = pl.program_id(0)
        p = i // nb
        row = (i % nb) * NBR

        @pl.when(i == 0)
        def _():
            st1_v[...] = jnp.zeros_like(st1_v)

        @pl.when(i == nb)
        def _():
            st2_v[...] = jnp.zeros_like(st2_v)

        @pl.when(i == 2 * nb)
        def _():
            st3_v[...] = jnp.zeros_like(st3_v)

        @pl.when(i == 3 * nb)
        def _():
            uv_v[...] = jnp.zeros_like(uv_v)

        oh = (gn_ref[...] == lax.broadcasted_iota(jnp.int32, (NBR, G), 1))

        @pl.when(p == 0)
        def _():
            a2 = a2a[...] + a2b[...]
            cnt = jnp.maximum(a2[:, 44], 1.0)
            inv = (1.0 / cnt)[:, None]
            agg = jnp.concatenate(
                [a0[...], a1[...], a2[:, 0:44]], axis=1) * inv
            h = jnp.dot(agg.astype(jnp.bfloat16), ua_ref[...],
                        preferred_element_type=jnp.float32)
            h += jnp.dot(n_ref[...].astype(jnp.bfloat16), ub_ref[...],
                         preferred_element_type=jnp.float32)
            h += jnp.dot(oh.astype(jnp.bfloat16), sn_ref[...],
                         preferred_element_type=jnp.float32)
            hb = h.astype(jnp.bfloat16)
            hv1_v[pl.ds(row, NBR), :] = hb
            acc_stats(st1_v, hb, 600)

        @pl.when(p == 1)
        def _():
            x = _norm_act(hv1_v[pl.ds(row, NBR), :].astype(jnp.float32),
                          st1_v[...], gb1_ref[...], N)
            h = jnp.dot(x.astype(jnp.bfloat16), w2_ref[...],
                        preferred_element_type=jnp.float32)
            hb = h.astype(jnp.bfloat16)
            hv2_v[pl.ds(row, NBR), :] = hb
            acc_stats(st2_v, hb, 600)

        @pl.when(p == 2)
        def _():
            x = _norm_act(hv2_v[pl.ds(row, NBR), :].astype(jnp.float32),
                          st2_v[...], gb2_ref[...], N)
            h = jnp.dot(x.astype(jnp.bfloat16), w3_ref[...],
                        preferred_element_type=jnp.float32)
            hb = h.astype(jnp.bfloat16)
            hv1_v[pl.ds(row, NBR), 0:D] = hb
            acc_stats(st3_v, hb, D)

        @pl.when(p == 3)
        def _():
            vip = n_ref[...] + _norm_act(
                hv1_v[pl.ds(row, NBR), 0:D].astype(jnp.float32),
                st3_v[...], gb3_ref[...], N)
            v_ref[...] = vip
            vx = jnp.concatenate(
                [vip.astype(jnp.bfloat16), jnp.ones((NBR, 1), jnp.bfloat16),
                 jnp.zeros((NBR, 3), jnp.bfloat16)], axis=1)
            uv_v[...] += lax.dot_general(
                oh.astype(jnp.bfloat16), vx, (((0,), (0,)), ((), ())),
                preferred_element_type=jnp.float32)

        @pl.when(i == 4 * nb - 1)
        def _():
            ue = ue_ref[:, :D] / jnp.maximum(ue_ref[:, D:D + 1], 1.0)
            uva = uv_v[...]
            uv = uva[:, :D] / jnp.maximum(uva[:, D:D + 1], 1.0)
            x = jnp.concatenate([ue, uv, s_ref[...]], axis=1)
            hu = jnp.dot(x.astype(jnp.bfloat16), wu0_ref[...],
                         preferred_element_type=jnp.float32)
            x = ubn_act(hu, gu0_ref[...])
            hu = jnp.dot(x.astype(jnp.bfloat16), wu1_ref[...],
                         preferred_element_type=jnp.float32)
            x = ubn_act(hu, gu1_ref[...])
            hu = jnp.dot(x.astype(jnp.bfloat16), wu2_ref[...],
                         preferred_element_type=jnp.float32)
            up_ref[...] = s_ref[...] + ubn_act(hu, gu2_ref[...])

    return pl.pallas_call(
        body,
        grid=grid,
        in_specs=[
            pl.BlockSpec((NBR, DH), jmap(0)),
            pl.BlockSpec((NBR, DH), jmap(0)),
            pl.BlockSpec((NBR, DH), jmap(0)),
            pl.BlockSpec((NBR, DH), jmap(0)),
            pl.BlockSpec((NBR, D), jmap(0, 3)),
            pl.BlockSpec((NBR, 1), jmap(0, 3)),
            _acc_spec((G, DU)), _acc_spec((G, D)),
            _acc_spec((D, 600)), _acc_spec((D, 600)), _acc_spec((G, 600)),
            _acc_spec((600, 600)), _acc_spec((600, D)),
            _acc_spec((8, 600)), _acc_spec((8, 600)), _acc_spec((8, D)),
            _acc_spec((900, 600)), _acc_spec((600, 600)), _acc_spec((600, D)),
            _acc_spec((8, 600)), _acc_spec((8, 600)), _acc_spec((8, D)),
        ],
        out_specs=[pl.BlockSpec((NBR, D), jmap(3)), _acc_spec((G, D))],
        out_shape=[
            jax.ShapeDtypeStruct((N, D), jnp.float32),
            jax.ShapeDtypeStruct((G, D), jnp.float32),
        ],
        scratch_shapes=[
            pltpu.VMEM((N, 600), jnp.bfloat16),
            pltpu.VMEM((N, 600), jnp.bfloat16),
            pltpu.VMEM((8, 600), jnp.float32),
            pltpu.VMEM((8, 600), jnp.float32),
            pltpu.VMEM((8, D), jnp.float32),
            pltpu.VMEM((G, DU), jnp.float32),
        ],
    )(acc0, acc1, acc2a, acc2b, nodes, gnode2d, ue_acc, states,
      ua, ub, sn, w2, w3, gb1, gb2, gb3, wu0, wu1, wu2, gu0, gu1, gu2)


def _gb(p):
    return jnp.stack([p["gamma"], p["beta"]] + [jnp.zeros_like(p["gamma"])] * 6)


def kernel(nodes, edges, states, params, index1, index2, gnode, gbond):
    f32 = jnp.float32
    bf16 = jnp.bfloat16
    index1 = index1.astype(jnp.int32)
    index2 = index2.astype(jnp.int32)
    gnode = gnode.astype(jnp.int32)
    gbond = gbond.astype(jnp.int32)

    nodes_bf = jnp.pad(nodes.astype(bf16), ((0, 0), (0, DP - D)))
    lo16 = lax.bitcast_convert_type(nodes_bf[:, :_GW], jnp.uint16)
    hi16 = lax.bitcast_convert_type(nodes_bf[:, _GW:], jnp.uint16)
    nodes_pk = lax.bitcast_convert_type(
        lo16.astype(jnp.uint32) | (hi16.astype(jnp.uint32) << 16), jnp.int32)
    i1_p1 = index1.reshape(16, -1, _SCC)
    i1_p2 = index1.reshape(32, -1, _SCC)
    gbond2d = gbond.reshape(E, 1)
    gnode2d = gnode.reshape(N, 1)
    zinit = jnp.zeros((NPAD, DH), f32)

    pe = params["mlp_e"]
    pv = params["mlp_v"]
    pu = params["mlp_u"]
    pa = params["edge_agg"]
    w0 = pe[0]["W"]
    wa = jnp.pad(w0[0:300], ((0, DP - D), (0, 0))).astype(bf16)
    wb = jnp.pad(w0[300:600], ((0, DP - D), (0, 0))).astype(bf16)
    wc = w0[600:900].astype(bf16)
    wd = w0[900:1200]
    u0 = pv[0]["W"]
    ua = u0[0:300].astype(bf16)
    ub = u0[300:600].astype(bf16)
    uc = u0[600:900]

    fs, fr = _sc_gather(nodes_pk, index1, index2)

    wd_uc = jnp.concatenate([wd, uc], axis=1).astype(bf16)
    h1, st1, sn = _tc_h1(fs, fr, edges, gbond2d, states, wa, wb, wc, wd_uc)
    h2, st2 = _tc_mm(h1, st1, _gb(pe[0]), pe[1]["W"].astype(bf16), E)
    h3, st3 = _tc_mm(h2, st2, _gb(pe[1]), pe[2]["W"].astype(bf16), E)

    e_k_p, h4, st4, ue_acc = _tc_ekp_h4(h3, st3, _gb(pe[2]), edges, gbond2d,
                                        pa[0]["W"].astype(bf16))
    h5, st5 = _tc_mm(h4, st4, _gb(pa[0]), pa[1]["W"].astype(bf16), E)
    et0, et1, et2 = _tc_et(h5, st5, _gb(pa[1]))

    acc0, acc1, acc2a, acc2b = _sc_scatter(et0, et1, et2, i1_p1, i1_p2, zinit)

    v_i_p, u_p = _tc_node_all(
        acc0, acc1, acc2a, acc2b, nodes, gnode2d, ue_acc, states,
        ua, ub, sn, pv[1]["W"].astype(bf16), pv[2]["W"].astype(bf16),
        _gb(pv[0]), _gb(pv[1]), _gb(pv[2]),
        pu[0]["W"].astype(bf16), pu[1]["W"].astype(bf16),
        pu[2]["W"].astype(bf16), _gb(pu[0]), _gb(pu[1]), _gb(pu[2]))

    return (v_i_p, e_k_p, u_p)


# speedup vs baseline: 1.1064x; 1.1064x over previous
import functools

import jax
import jax.numpy as jnp
from jax import lax
from jax.experimental import pallas as pl
from jax.experimental.pallas import tpu as pltpu
from jax.experimental.pallas import tpu_sc as plsc

N = 10000
E = 160000
G = 64
D = 300
DP = 512
DH = 128
DU = 304
NPAD = 10240
EBR = 3200
NBR = 400
EPS = 1e-5


def _softplus(x):
    return jnp.log1p(jnp.minimum(jnp.exp(x), jnp.float32(1e38)))


def _norm_act(h, stats, gb, nrows):
    s = stats[0, :]
    ss = stats[1, :]
    mean = s / nrows
    var = ss / nrows - mean * mean
    a = lax.rsqrt(var + EPS) * gb[0, :]
    b = gb[1, :] - mean * a
    return _softplus(h * a[None, :] + b[None, :])


def _stats_update(stats_ref, hb, step):
    @pl.when(step == 0)
    def _():
        stats_ref[...] = jnp.zeros_like(stats_ref)

    ones8 = jnp.ones((8, hb.shape[0]), jnp.bfloat16)
    ps = jnp.dot(ones8, hb, preferred_element_type=jnp.float32)
    pss = jnp.dot(ones8, hb * hb, preferred_element_type=jnp.float32)
    pad = jnp.zeros((6, hb.shape[1]), jnp.float32)
    stats_ref[...] += jnp.concatenate([ps[0:1, :], pss[0:1, :], pad], axis=0)


def _rows(br):
    return pl.BlockSpec((br, None), lambda i: (i, 0))


def _full(shape):
    return pl.BlockSpec(shape, lambda i: tuple(0 for _ in shape))


def _acc_spec(shape):
    return pl.BlockSpec(shape, lambda i: tuple(0 for _ in shape))


def _rows_spec(br, ncols):
    return pl.BlockSpec((br, ncols), lambda i: (i, 0))


_GC = 128
_GW = 256


def _sc_gather(table, idx1, idx2):
    nw = 32
    bpw = E // nw
    nfull = bpw // _GC
    npair = nfull // 2
    tail = bpw - nfull * _GC
    mesh = plsc.VectorSubcoreMesh(core_axis_name="c", subcore_axis_name="s")

    @functools.partial(
        pl.kernel,
        mesh=mesh,
        out_type=[
            jax.ShapeDtypeStruct((E, _GW), jnp.int32),
            jax.ShapeDtypeStruct((E, _GW), jnp.int32),
        ],
        scratch_types=[
            pltpu.VMEM((bpw,), jnp.int32),
            pltpu.VMEM((_GC, _GW), jnp.int32),
            pltpu.VMEM((_GC, _GW), jnp.int32),
            pltpu.SemaphoreType.DMA,
            pltpu.SemaphoreType.DMA,
            pltpu.SemaphoreType.DMA,
            pltpu.SemaphoreType.DMA,
        ],
    )
    def k(tab_hbm, i1_hbm, i2_hbm, o1_hbm, o2_hbm,
          idx_v, b0, b1, sg0, sg1, sw0, sw1):
        wid = lax.axis_index("s") * 2 + lax.axis_index("c")
        base = wid * bpw

        def one_table(i_hbm, o_hbm):
            pltpu.sync_copy(i_hbm.at[pl.ds(base, bpw)], idx_v)

            def pair(jj, _):
                j0 = jj * 2
                g0 = pltpu.async_copy(
                    tab_hbm.at[idx_v.at[pl.ds(j0 * _GC, _GC)]], b0, sg0)
                g1 = pltpu.async_copy(
                    tab_hbm.at[idx_v.at[pl.ds((j0 + 1) * _GC, _GC)]], b1, sg1)
                g0.wait()
                w0 = pltpu.async_copy(
                    b0, o_hbm.at[pl.ds(base + j0 * _GC, _GC)], sw0)
                g1.wait()
                w1 = pltpu.async_copy(
                    b1, o_hbm.at[pl.ds(base + (j0 + 1) * _GC, _GC)], sw1)
                w0.wait()
                w1.wait()
                return 0

            lax.fori_loop(0, npair, pair, 0)
            j = nfull - 1
            pltpu.async_copy(
                tab_hbm.at[idx_v.at[pl.ds(j * _GC, _GC)]], b0, sg0).wait()
            w0 = pltpu.async_copy(
                b0, o_hbm.at[pl.ds(base + j * _GC, _GC)], sw0)
            pltpu.async_copy(
                tab_hbm.at[idx_v.at[pl.ds(nfull * _GC, tail)]],
                b1.at[pl.ds(0, tail)], sg1).wait()
            pltpu.sync_copy(b1.at[pl.ds(0, tail)],
                            o_hbm.at[pl.ds(base + nfull * _GC, tail)])
            w0.wait()

        one_table(i1_hbm, o1_hbm)
        one_table(i2_hbm, o2_hbm)

    return k(table, idx1, idx2)


_SCC = 40


def _sc_scatter(et0, et1, et2, idx_p1, idx_p2, zeros_init):
    nt = 16
    epw = E // nt
    nch1 = epw // _SCC
    npr1 = nch1 // 2
    e2w = E // 32
    nch2 = e2w // _SCC
    npr2 = nch2 // 2
    npt = NPAD // nt
    mesh = plsc.VectorSubcoreMesh(core_axis_name="c", subcore_axis_name="s")

    @functools.partial(
        pl.kernel,
        mesh=mesh,
        out_type=[
            jax.ShapeDtypeStruct((NPAD, DH), jnp.float32),
            jax.ShapeDtypeStruct((NPAD, DH), jnp.float32),
            jax.ShapeDtypeStruct((NPAD, DH), jnp.float32),
            jax.ShapeDtypeStruct((NPAD, DH), jnp.float32),
        ],
        scratch_types=[
            pltpu.VMEM((nch1, _SCC), jnp.int32),
            pltpu.VMEM((_SCC, DH), jnp.float32),
            pltpu.VMEM((_SCC, DH), jnp.float32),
            pltpu.VMEM_SHARED((NPAD, DH), jnp.float32),
            pltpu.SemaphoreType.DMA,
            pltpu.SemaphoreType.DMA,
        ],
    )
    def k(e0_hbm, e1_hbm, e2_hbm, ip1_hbm, ip2_hbm, z_hbm,
          o0_hbm, o1_hbm, o2a_hbm, o2b_hbm,
          idx_v, b0, b1, acc, s0, s1):
        cid = lax.axis_index("c")
        sid = lax.axis_index("s")
        r0 = sid * npt
        pltpu.sync_copy(ip1_hbm.at[sid], idx_v)
        pltpu.sync_copy(z_hbm.at[pl.ds(r0, npt)], acc.at[pl.ds(r0, npt)])
        plsc.subcore_barrier()

        def pairs(src_hbm, base, npairs):
            def pair(jj, _):
                j0 = jj * 2
                off0 = base + j0 * _SCC
                l0 = pltpu.async_copy(src_hbm.at[pl.ds(off0, _SCC)], b0, s0)
                l1 = pltpu.async_copy(
                    src_hbm.at[pl.ds(off0 + _SCC, _SCC)], b1, s1)
                l0.wait()
                pltpu.sync_copy(b0, acc.at[idx_v.at[j0]], add=True)
                l1.wait()
                pltpu.sync_copy(b1, acc.at[idx_v.at[j0 + 1]], add=True)
                return 0

            lax.fori_loop(0, npairs, pair, 0)

        @pl.when(cid == 0)
        def _():
            pairs(e0_hbm, sid * epw, npr1)

        @pl.when(cid == 1)
        def _():
            pairs(e1_hbm, sid * epw, npr1)

        plsc.subcore_barrier()

        @pl.when(cid == 0)
        def _():
            pltpu.sync_copy(acc.at[pl.ds(r0, npt)], o0_hbm.at[pl.ds(r0, npt)])

        @pl.when(cid == 1)
        def _():
            pltpu.sync_copy(acc.at[pl.ds(r0, npt)], o1_hbm.at[pl.ds(r0, npt)])

        plsc.subcore_barrier()
        pltpu.sync_copy(ip2_hbm.at[cid * nt + sid], idx_v.at[pl.ds(0, nch2)])
        pltpu.sync_copy(z_hbm.at[pl.ds(r0, npt)], acc.at[pl.ds(r0, npt)])
        plsc.subcore_barrier()

        base2 = cid * (E // 2) + sid * e2w
        pairs(e2_hbm, base2, npr2)
        j = nch2 - 1
        pltpu.sync_copy(e2_hbm.at[pl.ds(base2 + j * _SCC, _SCC)], b0)
        pltpu.sync_copy(b0, acc.at[idx_v.at[j]], add=True)
        plsc.subcore_barrier()

        @pl.when(cid == 0)
        def _():
            pltpu.sync_copy(acc.at[pl.ds(r0, npt)], o2a_hbm.at[pl.ds(r0, npt)])

        @pl.when(cid == 1)
        def _():
            pltpu.sync_copy(acc.at[pl.ds(r0, npt)], o2b_hbm.at[pl.ds(r0, npt)])

    return k(et0, et1, et2, idx_p1, idx_p2, zeros_init)


def _tc_h1(fs, fr, edges, gbond2d, states, wa, wb, wc, wd_uc):
    grid = (E // EBR,)

    def body(fs_ref, fr_ref, e_ref, gb_ref, s_ref, wa_ref, wb_ref, wc_ref,
             wduc_ref, h_ref, st_ref, sn_ref, sb_v):
        i = pl.program_id(0)

        @pl.when(i == 0)
        def _():
            sbsn = jnp.dot(s_ref[...].astype(jnp.bfloat16), wduc_ref[...],
                           preferred_element_type=jnp.float32)
            sb_v[...] = sbsn[:, :600].astype(jnp.bfloat16)
            sn_ref[...] = sbsn[:, 600:].astype(jnp.bfloat16)

        oh = (gb_ref[...] == lax.broadcasted_iota(jnp.int32, (EBR, G), 1))
        h = jnp.dot(e_ref[...].astype(jnp.bfloat16), wc_ref[...],
                    preferred_element_type=jnp.float32)
        h += jnp.dot(oh.astype(jnp.bfloat16), sb_v[...],
                     preferred_element_type=jnp.float32)
        for x_ref, w_ref in ((fs_ref, wa_ref), (fr_ref, wb_ref)):
            x_i = x_ref[...]
            x_lo = lax.bitcast_convert_type(
                jnp.left_shift(x_i, 16), jnp.float32)
            x_hi = lax.bitcast_convert_type(
                jnp.bitwise_and(x_i[:, 0:128], jnp.int32(-65536)), jnp.float32)
            h += jnp.dot(x_lo.astype(jnp.bfloat16), w_ref[pl.ds(0, _GW), :],
                         preferred_element_type=jnp.float32)
            h += jnp.dot(x_hi.astype(jnp.bfloat16), w_ref[pl.ds(_GW, 128), :],
                         preferred_element_type=jnp.float32)
        hb = h.astype(jnp.bfloat16)
        h_ref[...] = hb
        _stats_update(st_ref, hb, i)

    return pl.pallas_call(
        body,
        grid=grid,
        in_specs=[
            _rows_spec(EBR, _GW), _rows_spec(EBR, _GW),
            _rows_spec(EBR, D),
            _rows_spec(EBR, 1),
            _acc_spec((G, D)),
            _acc_spec((DP, 600)), _acc_spec((DP, 600)), _acc_spec((D, 600)),
            _acc_spec((D, 1200)),
        ],
        out_specs=[_rows_spec(EBR, 600), _acc_spec((8, 600)),
                   _acc_spec((G, 600))],
        out_shape=[
            jax.ShapeDtypeStruct((E, 600), jnp.bfloat16),
            jax.ShapeDtypeStruct((8, 600), jnp.float32),
            jax.ShapeDtypeStruct((G, 600), jnp.bfloat16),
        ],
        scratch_shapes=[pltpu.VMEM((G, 600), jnp.bfloat16)],
    )(fs, fr, edges, gbond2d, states, wa, wb, wc, wd_uc)


def _tc_mm(h_prev, stats, gb, w, nrows):
    rtot, kdim = h_prev.shape
    ndim = w.shape[1]
    br = EBR if rtot == E else NBR
    grid = (rtot // br,)

    def body(hp_ref, st_in_ref, gb_ref, w_ref, h_ref, st_ref):
        i = pl.program_id(0)
        x = _norm_act(hp_ref[...].astype(jnp.float32), st_in_ref[...],
                      gb_ref[...], nrows)
        h = jnp.dot(x.astype(jnp.bfloat16), w_ref[...],
                    preferred_element_type=jnp.float32)
        hb = h.astype(jnp.bfloat16)
        h_ref[...] = hb
        _stats_update(st_ref, hb, i)

    return pl.pallas_call(
        body,
        grid=grid,
        in_specs=[
            _rows_spec(br, kdim), _acc_spec((8, kdim)), _acc_spec((8, kdim)),
            _acc_spec((kdim, ndim)),
        ],
        out_specs=[_rows_spec(br, ndim), _acc_spec((8, ndim))],
        out_shape=[
            jax.ShapeDtypeStruct((rtot, ndim), jnp.bfloat16),
            jax.ShapeDtypeStruct((8, ndim), jnp.float32),
        ],
    )(h_prev, stats, gb, w)


def _tc_ekp_h4(h3, stats3, gb3, edges, gbond2d, v0):
    grid = (E // EBR,)

    def body(h3_ref, st3_ref, gb3_ref, e_ref, gbd_ref, v0_ref,
             ekp_ref, h4_ref, st4_ref, ue_ref):
        i = pl.program_id(0)
        ek = e_ref[...] + _norm_act(h3_ref[...].astype(jnp.float32),
                                    st3_ref[...], gb3_ref[...], E)
        ekp_ref[...] = ek
        ekb = ek.astype(jnp.bfloat16)
        h4 = jnp.dot(ekb, v0_ref[...], preferred_element_type=jnp.float32)
        h4b = h4.astype(jnp.bfloat16)
        h4_ref[...] = h4b
        _stats_update(st4_ref, h4b, i)
        oh = (gbd_ref[...] == lax.broadcasted_iota(jnp.int32, (EBR, G), 1))
        ekx = jnp.concatenate(
            [ekb, jnp.ones((EBR, 1), jnp.bfloat16),
             jnp.zeros((EBR, 3), jnp.bfloat16)], axis=1)

        @pl.when(i == 0)
        def _():
            ue_ref[...] = jnp.zeros_like(ue_ref)

        ue_ref[...] += lax.dot_general(
            oh.astype(jnp.bfloat16), ekx, (((0,), (0,)), ((), ())),
            preferred_element_type=jnp.float32)

    return pl.pallas_call(
        body,
        grid=grid,
        in_specs=[
            _rows_spec(EBR, D), _acc_spec((8, D)), _acc_spec((8, D)),
            _rows_spec(EBR, D), _rows_spec(EBR, 1), _acc_spec((D, 600)),
        ],
        out_specs=[_rows_spec(EBR, D), _rows_spec(EBR, 600),
                   _acc_spec((8, 600)), _acc_spec((G, DU))],
        out_shape=[
            jax.ShapeDtypeStruct((E, D), jnp.float32),
            jax.ShapeDtypeStruct((E, 600), jnp.bfloat16),
            jax.ShapeDtypeStruct((8, 600), jnp.float32),
            jax.ShapeDtypeStruct((G, DU), jnp.float32),
        ],
    )(h3, stats3, gb3, edges, gbond2d, v0)


def _tc_et(h5, stats5, gb5):
    grid = (E // EBR,)

    def body(h5_ref, st_ref, gb_ref, s0_ref, s1_ref, s2_ref):
        et = _norm_act(h5_ref[...].astype(jnp.float32), st_ref[...],
                       gb_ref[...], E)
        one = jnp.ones((EBR, 1), jnp.float32)
        zpad = jnp.zeros((EBR, 83), jnp.float32)
        s0_ref[...] = et[:, 0:128]
        s1_ref[...] = et[:, 128:256]
        s2_ref[...] = jnp.concatenate([et[:, 256:300], one, zpad], axis=1)

    return pl.pallas_call(
        body,
        grid=grid,
        in_specs=[_rows_spec(EBR, D), _acc_spec((8, D)), _acc_spec((8, D))],
        out_specs=[_rows_spec(EBR, DH), _rows_spec(EBR, DH),
                   _rows_spec(EBR, DH)],
        out_shape=[
            jax.ShapeDtypeStruct((E, DH), jnp.float32),
            jax.ShapeDtypeStruct((E, DH), jnp.float32),
            jax.ShapeDtypeStruct((E, DH), jnp.float32),
        ],
    )(h5, stats5, gb5)


def _tc_node_all(acc0, acc1, acc2a, acc2b, nodes, gnode2d, ue_acc, states,
                 ua, ub, sn, w2, w3, gb1, gb2, gb3,
                 wu0, wu1, wu2, gu0, gu1, gu2):
    nb = N // NBR
    grid = (4 * nb,)

    def jmap(*phases):
        def f(i):
            p = i // nb
            use = functools.reduce(jnp.logical_or,
                                   [p == q for q in phases])
            return (jnp.where(use, i % nb, 0), 0)
        return f

    def acc_stats(st_v, hb, w):
        ones8 = jnp.ones((8, NBR), jnp.bfloat16)
        ps = jnp.dot(ones8, hb, preferred_element_type=jnp.float32)
        pss = jnp.dot(ones8, hb * hb, preferred_element_type=jnp.float32)
        st_v[...] += jnp.concatenate(
            [ps[0:1, :], pss[0:1, :], jnp.zeros((6, w), jnp.float32)], axis=0)

    def ubn_act(h, gb):
        mean = jnp.mean(h, axis=0)
        var = jnp.mean(h * h, axis=0) - mean * mean
        a = lax.rsqrt(var + EPS) * gb[0:1, :]
        return _softplus(h * a + (gb[1:2, :] - mean[None, :] * a))

    def body(a0, a1, a2a, a2b, n_ref, gn_ref, ue_ref, s_ref,
             ua_ref, ub_ref, sn_ref, w2_ref, w3_ref,
             gb1_ref, gb2_ref, gb3_ref,
             wu0_ref, wu1_ref, wu2_ref, gu0_ref, gu1_ref, gu2_ref,
             v_ref, up_ref,
             hv1_v, hv2_v, st1_v, st2_v, st3_v, uv_v):
        i = pl.program_id(0)
        p = i // nb
        row = (i % nb) * NBR

        @pl.when(i == 0)
        def _():
            st1_v[...] = jnp.zeros_like(st1_v)

        @pl.when(i == nb)
        def _():
            st2_v[...] = jnp.zeros_like(st2_v)

        @pl.when(i == 2 * nb)
        def _():
            st3_v[...] = jnp.zeros_like(st3_v)

        @pl.when(i == 3 * nb)
        def _():
            uv_v[...] = jnp.zeros_like(uv_v)

        oh = (gn_ref[...] == lax.broadcasted_iota(jnp.int32, (NBR, G), 1))

        @pl.when(p == 0)
        def _():
            a2 = a2a[...] + a2b[...]
            cnt = jnp.maximum(a2[:, 44], 1.0)
            inv = (1.0 / cnt)[:, None]
            agg = jnp.concatenate(
                [a0[...], a1[...], a2[:, 0:44]], axis=1) * inv
            h = jnp.dot(agg.astype(jnp.bfloat16), ua_ref[...],
                        preferred_element_type=jnp.float32)
            h += jnp.dot(n_ref[...].astype(jnp.bfloat16), ub_ref[...],
                         preferred_element_type=jnp.float32)
            h += jnp.dot(oh.astype(jnp.bfloat16), sn_ref[...],
                         preferred_element_type=jnp.float32)
            hb = h.astype(jnp.bfloat16)
            hv1_v[pl.ds(row, NBR), :] = hb
            acc_stats(st1_v, hb, 600)

        @pl.when(p == 1)
        def _():
            x = _norm_act(hv1_v[pl.ds(row, NBR), :].astype(jnp.float32),
                          st1_v[...], gb1_ref[...], N)
            h = jnp.dot(x.astype(jnp.bfloat16), w2_ref[...],
                        preferred_element_type=jnp.float32)
            hb = h.astype(jnp.bfloat16)
            hv2_v[pl.ds(row, NBR), :] = hb
            acc_stats(st2_v, hb, 600)

        @pl.when(p == 2)
        def _():
            x = _norm_act(hv2_v[pl.ds(row, NBR), :].astype(jnp.float32),
                          st2_v[...], gb2_ref[...], N)
            h = jnp.dot(x.astype(jnp.bfloat16), w3_ref[...],
                        preferred_element_type=jnp.float32)
            hb = h.astype(jnp.bfloat16)
            hv1_v[pl.ds(row, NBR), 0:D] = hb
            acc_stats(st3_v, hb, D)

        @pl.when(p == 3)
        def _():
            vip = n_ref[...] + _norm_act(
                hv1_v[pl.ds(row, NBR), 0:D].astype(jnp.float32),
                st3_v[...], gb3_ref[...], N)
            v_ref[...] = vip
            vx = jnp.concatenate(
                [vip.astype(jnp.bfloat16), jnp.ones((NBR, 1), jnp.bfloat16),
                 jnp.zeros((NBR, 3), jnp.bfloat16)], axis=1)
            uv_v[...] += lax.dot_general(
                oh.astype(jnp.bfloat16), vx, (((0,), (0,)), ((), ())),
                preferred_element_type=jnp.float32)

        @pl.when(i == 4 * nb - 1)
        def _():
            ue = ue_ref[:, :D] / jnp.maximum(ue_ref[:, D:D + 1], 1.0)
            uva = uv_v[...]
            uv = uva[:, :D] / jnp.maximum(uva[:, D:D + 1], 1.0)
            x = jnp.concatenate([ue, uv, s_ref[...]], axis=1)
            hu = jnp.dot(x.astype(jnp.bfloat16), wu0_ref[...],
                         preferred_element_type=jnp.float32)
            x = ubn_act(hu, gu0_ref[...])
            hu = jnp.dot(x.astype(jnp.bfloat16), wu1_ref[...],
                         preferred_element_type=jnp.float32)
            x = ubn_act(hu, gu1_ref[...])
            hu = jnp.dot(x.astype(jnp.bfloat16), wu2_ref[...],
                         preferred_element_type=jnp.float32)
            up_ref[...] = s_ref[...] + ubn_act(hu, gu2_ref[...])

    return pl.pallas_call(
        body,
        grid=grid,
        in_specs=[
            pl.BlockSpec((NBR, DH), jmap(0)),
            pl.BlockSpec((NBR, DH), jmap(0)),
            pl.BlockSpec((NBR, DH), jmap(0)),
            pl.BlockSpec((NBR, DH), jmap(0)),
            pl.BlockSpec((NBR, D), jmap(0, 3)),
            pl.BlockSpec((NBR, 1), jmap(0, 3)),
            _acc_spec((G, DU)), _acc_spec((G, D)),
            _acc_spec((D, 600)), _acc_spec((D, 600)), _acc_spec((G, 600)),
            _acc_spec((600, 600)), _acc_spec((600, D)),
            _acc_spec((8, 600)), _acc_spec((8, 600)), _acc_spec((8, D)),
            _acc_spec((900, 600)), _acc_spec((600, 600)), _acc_spec((600, D)),
            _acc_spec((8, 600)), _acc_spec((8, 600)), _acc_spec((8, D)),
        ],
        out_specs=[pl.BlockSpec((NBR, D), jmap(3)), _acc_spec((G, D))],
        out_shape=[
            jax.ShapeDtypeStruct((N, D), jnp.float32),
            jax.ShapeDtypeStruct((G, D), jnp.float32),
        ],
        scratch_shapes=[
            pltpu.VMEM((N, 600), jnp.bfloat16),
            pltpu.VMEM((N, 600), jnp.bfloat16),
            pltpu.VMEM((8, 600), jnp.float32),
            pltpu.VMEM((8, 600), jnp.float32),
            pltpu.VMEM((8, D), jnp.float32),
            pltpu.VMEM((G, DU), jnp.float32),
        ],
    )(acc0, acc1, acc2a, acc2b, nodes, gnode2d, ue_acc, states,
      ua, ub, sn, w2, w3, gb1, gb2, gb3, wu0, wu1, wu2, gu0, gu1, gu2)


def _gb(p):
    return jnp.stack([p["gamma"], p["beta"]] + [jnp.zeros_like(p["gamma"])] * 6)


def kernel(nodes, edges, states, params, index1, index2, gnode, gbond):
    f32 = jnp.float32
    bf16 = jnp.bfloat16
    index1 = index1.astype(jnp.int32)
    index2 = index2.astype(jnp.int32)
    gnode = gnode.astype(jnp.int32)
    gbond = gbond.astype(jnp.int32)

    nodes_bf = jnp.pad(nodes.astype(bf16), ((0, 0), (0, DP - D)))
    lo16 = lax.bitcast_convert_type(nodes_bf[:, :_GW], jnp.uint16)
    hi16 = lax.bitcast_convert_type(nodes_bf[:, _GW:], jnp.uint16)
    nodes_pk = lax.bitcast_convert_type(
        lo16.astype(jnp.uint32) | (hi16.astype(jnp.uint32) << 16), jnp.int32)
    i1_p1 = index1.reshape(16, -1, _SCC)
    i1_p2 = index1.reshape(32, -1, _SCC)
    gbond2d = gbond.reshape(E, 1)
    gnode2d = gnode.reshape(N, 1)
    zinit = jnp.zeros((NPAD, DH), f32)

    pe = params["mlp_e"]
    pv = params["mlp_v"]
    pu = params["mlp_u"]
    pa = params["edge_agg"]
    w0 = pe[0]["W"]
    wa = jnp.pad(w0[0:300], ((0, DP - D), (0, 0))).astype(bf16)
    wb = jnp.pad(w0[300:600], ((0, DP - D), (0, 0))).astype(bf16)
    wc = w0[600:900].astype(bf16)
    wd = w0[900:1200]
    u0 = pv[0]["W"]
    ua = u0[0:300].astype(bf16)
    ub = u0[300:600].astype(bf16)
    uc = u0[600:900]

    fs, fr = _sc_gather(nodes_pk, index1, index2)

    wd_uc = jnp.concatenate([wd, uc], axis=1).astype(bf16)
    h1, st1, sn = _tc_h1(fs, fr, edges, gbond2d, states, wa, wb, wc, wd_uc)
    h2, st2 = _tc_mm(h1, st1, _gb(pe[0]), pe[1]["W"].astype(bf16), E)
    h3, st3 = _tc_mm(h2, st2, _gb(pe[1]), pe[2]["W"].astype(bf16), E)

    e_k_p, h4, st4, ue_acc = _tc_ekp_h4(h3, st3, _gb(pe[2]), edges, gbond2d,
                                        pa[0]["W"].astype(bf16))
    h5, st5 = _tc_mm(h4, st4, _gb(pa[0]), pa[1]["W"].astype(bf16), E)
    et0, et1, et2 = _tc_et(h5, st5, _gb(pa[1]))

    acc0, acc1, acc2a, acc2b = _sc_scatter(et0, et1, et2, i1_p1, i1_p2, zinit)

    v_i_p, u_p = _tc_node_all(
        acc0, acc1, acc2a, acc2b, nodes, gnode2d, ue_acc, states,
        ua, ub, sn, pv[1]["W"].astype(bf16), pv[2]["W"].astype(bf16),
        _gb(pv[0]), _gb(pv[1]), _gb(pv[2]),
        pu[0]["W"].astype(bf16), pu[1]["W"].astype(bf16),
        pu[2]["W"].astype(bf16), _gb(pu[0]), _gb(pu[1]), _gb(pu[2]))

    return (v_i_p, e_k_p, u_p)
